# Initial kernel scaffold; baseline (speedup 1.0000x reference)
import jax
import jax.numpy as jnp
from jax.experimental import pallas as pl


def kernel(x, edge_index, W_in, b_in, W1, b1, W2, b2, W3, b3, W_out, b_out):
    raise NotImplementedError("write your pallas kernel here")


# speedup vs baseline: 6.3053x; 6.3053x over previous
import functools

import jax
import jax.numpy as jnp
from jax import lax
from jax.experimental import pallas as pl
from jax.experimental.pallas import tpu as pltpu
from jax.experimental.pallas import tpu_sc as plsc

N = 10000
E = 320000
IN_DIM = 128
MP = 256
HALF = 128
OUT = 128

NT = 16
NC = 2
CH = 128
EPT = 20096
E_PAD = EPT * NT
N_PAD = N + 8
ROW_CH = 200
N_RCH = 5


def _sc_mesh():
    return plsc.VectorSubcoreMesh(core_axis_name="c", subcore_axis_name="s")


def _deg_body(colp_hbm, deg_hbm, acc_sh, ones_v, idx_v, buf_v):
    c = lax.axis_index("c")
    s = lax.axis_index("s")

    def fill_ones(i, _):
        ones_v[pl.ds(i * 16, 16)] = jnp.ones((16,), jnp.float32)
        return 0
    lax.fori_loop(0, 64, fill_ones, 0)

    @pl.when(c == 0)
    def _():
        @pl.when(s < 10)
        def _():
            pltpu.sync_copy(ones_v.at[pl.ds(0, 1000)], acc_sh.at[pl.ds(s * 1000, 1000)])

        @pl.when(s == 10)
        def _():
            pltpu.sync_copy(ones_v.at[pl.ds(0, 8)], acc_sh.at[pl.ds(N, 8)])

    plsc.subcore_barrier()

    @pl.when(c == 0)
    def _():
        def step(i, _):
            base = s * EPT + i * CH
            pltpu.sync_copy(colp_hbm.at[pl.ds(base, CH)], idx_v)
            pltpu.sync_copy(ones_v.at[pl.ds(0, CH)], acc_sh.at[idx_v], add=True)
            return 0
        lax.fori_loop(0, EPT // CH, step, 0)

    plsc.subcore_barrier()

    @pl.when((c == 0) & (s < 10))
    def _():
        pltpu.sync_copy(acc_sh.at[pl.ds(s * 1000, 1000)], buf_v)
        pltpu.sync_copy(buf_v, deg_hbm.at[pl.ds(s * 1000, 1000)])


def _deg_call(colp):
    f = pl.kernel(
        _deg_body,
        out_type=jax.ShapeDtypeStruct((N,), jnp.float32),
        mesh=_sc_mesh(),
        scratch_types=[
            pltpu.VMEM_SHARED((N_PAD,), jnp.float32),
            pltpu.VMEM((1024,), jnp.float32),
            pltpu.VMEM((CH,), jnp.int32),
            pltpu.VMEM((1000,), jnp.float32),
        ],
    )
    return f(colp)


def _scatter_body(g0_hbm, g1_hbm, rowp_hbm, colp_hbm, out0_hbm, out1_hbm,
                  acc_sh, idxr_v, idxc_v, rows_v, buf_v):
    c = lax.axis_index("c")
    s = lax.axis_index("s")

    @pl.when(s < 10)
    def _():
        def init(k, _):
            r0 = s * 1000 + k * ROW_CH

            @pl.when(c == 0)
            def _():
                pltpu.sync_copy(g0_hbm.at[pl.ds(r0, ROW_CH)], buf_v)

            @pl.when(c == 1)
            def _():
                pltpu.sync_copy(g1_hbm.at[pl.ds(r0, ROW_CH)], buf_v)

            pltpu.sync_copy(buf_v, acc_sh.at[pl.ds(r0, ROW_CH)])
            return 0
        lax.fori_loop(0, N_RCH, init, 0)

    plsc.subcore_barrier()

    def step(i, _):
        base = s * EPT + i * CH
        pltpu.sync_copy(rowp_hbm.at[pl.ds(base, CH)], idxr_v)
        pltpu.sync_copy(colp_hbm.at[pl.ds(base, CH)], idxc_v)

        @pl.when(c == 0)
        def _():
            pltpu.sync_copy(g0_hbm.at[idxr_v], rows_v)

        @pl.when(c == 1)
        def _():
            pltpu.sync_copy(g1_hbm.at[idxr_v], rows_v)

        pltpu.sync_copy(rows_v, acc_sh.at[idxc_v], add=True)
        return 0
    lax.fori_loop(0, EPT // CH, step, 0)

    plsc.subcore_barrier()

    @pl.when(s < 10)
    def _():
        def wb(k, _):
            r0 = s * 1000 + k * ROW_CH
            pltpu.sync_copy(acc_sh.at[pl.ds(r0, ROW_CH)], buf_v)

            @pl.when(c == 0)
            def _():
                pltpu.sync_copy(buf_v, out0_hbm.at[pl.ds(r0, ROW_CH)])

            @pl.when(c == 1)
            def _():
                pltpu.sync_copy(buf_v, out1_hbm.at[pl.ds(r0, ROW_CH)])

            return 0
        lax.fori_loop(0, N_RCH, wb, 0)


def _scatter_call(g0, g1, rowp, colp):
    f = pl.kernel(
        _scatter_body,
        out_type=[
            jax.ShapeDtypeStruct((N, HALF), jnp.float32),
            jax.ShapeDtypeStruct((N, HALF), jnp.float32),
        ],
        mesh=_sc_mesh(),
        scratch_types=[
            pltpu.VMEM_SHARED((N_PAD, HALF), jnp.float32),
            pltpu.VMEM((CH,), jnp.int32),
            pltpu.VMEM((CH,), jnp.int32),
            pltpu.VMEM((CH, HALF), jnp.float32),
            pltpu.VMEM((ROW_CH, HALF), jnp.float32),
        ],
    )
    return f(g0, g1, rowp, colp)


BLK = 1000


def _gelu(x):
    return x * 0.5 * (1.0 + lax.erf(x * 0.7071067811865476))


def _in_body(x_ref, w_ref, b_ref, deg_ref, g0_ref, g1_ref):
    h = jnp.dot(x_ref[...], w_ref[...], preferred_element_type=jnp.float32)
    h = h + b_ref[...]
    g = _gelu(h) / deg_ref[:, 0:1]
    g0_ref[...] = g[:, :HALF]
    g1_ref[...] = g[:, HALF:]


def _mid_body(a0_ref, a1_ref, w_ref, b_ref, deg_ref, g0_ref, g1_ref):
    h = jnp.dot(a0_ref[...], w_ref[:HALF, :], preferred_element_type=jnp.float32)
    h = h + jnp.dot(a1_ref[...], w_ref[HALF:, :], preferred_element_type=jnp.float32)
    h = h + b_ref[...]
    g = _gelu(h) / deg_ref[:, 0:1]
    g0_ref[...] = g[:, :HALF]
    g1_ref[...] = g[:, HALF:]


def _out_body(a0_ref, a1_ref, w_ref, b_ref, wo_ref, bo_ref, o_ref):
    h = jnp.dot(a0_ref[...], w_ref[:HALF, :], preferred_element_type=jnp.float32)
    h = h + jnp.dot(a1_ref[...], w_ref[HALF:, :], preferred_element_type=jnp.float32)
    h = h + b_ref[...]
    o_ref[...] = jnp.dot(h, wo_ref[...], preferred_element_type=jnp.float32) + bo_ref[...]


def _block(shape):
    return pl.BlockSpec(shape, lambda i: (0,) * len(shape))


def _rows(shape):
    return pl.BlockSpec(shape, lambda i: (i,) + (0,) * (len(shape) - 1))


def _in_call(x, w, b2, deg8):
    return pl.pallas_call(
        _in_body,
        grid=(N // BLK,),
        in_specs=[_rows((BLK, IN_DIM)), _block((IN_DIM, MP)), _block((1, MP)),
                  _rows((BLK, 8))],
        out_specs=[_rows((BLK, HALF)), _rows((BLK, HALF))],
        out_shape=[jax.ShapeDtypeStruct((N, HALF), jnp.float32)] * 2,
    )(x, w, b2, deg8)


def _mid_call(a0, a1, w, b2, deg8):
    return pl.pallas_call(
        _mid_body,
        grid=(N // BLK,),
        in_specs=[_rows((BLK, HALF)), _rows((BLK, HALF)), _block((MP, MP)),
                  _block((1, MP)), _rows((BLK, 8))],
        out_specs=[_rows((BLK, HALF)), _rows((BLK, HALF))],
        out_shape=[jax.ShapeDtypeStruct((N, HALF), jnp.float32)] * 2,
    )(a0, a1, w, b2, deg8)


def _out_call(a0, a1, w, b2, wo, bo2):
    return pl.pallas_call(
        _out_body,
        grid=(N // BLK,),
        in_specs=[_rows((BLK, HALF)), _rows((BLK, HALF)), _block((MP, MP)),
                  _block((1, MP)), _block((MP, OUT)), _block((1, OUT))],
        out_specs=_rows((BLK, OUT)),
        out_shape=jax.ShapeDtypeStruct((N, OUT), jnp.float32),
    )(a0, a1, w, b2, wo, bo2)


def kernel(x, edge_index, W_in, b_in, W1, b1, W2, b2, W3, b3, W_out, b_out):
    npad = E_PAD - E
    pad_row = (jnp.arange(npad, dtype=jnp.int32) % NT) * 625
    pad_col = N + (jnp.arange(npad, dtype=jnp.int32) % 8)
    rowp = jnp.concatenate([edge_index[0], pad_row])
    colp = jnp.concatenate([edge_index[1], pad_col])

    deg = _deg_call(colp)
    deg8 = jnp.broadcast_to(deg[:, None], (N, 8))

    g0, g1 = _in_call(x, W_in, b_in.reshape(1, MP), deg8)
    for Wl, bl in ((W1, b1), (W2, b2)):
        a0, a1 = _scatter_call(g0, g1, rowp, colp)
        g0, g1 = _mid_call(a0, a1, Wl, bl.reshape(1, MP), deg8)
    a0, a1 = _scatter_call(g0, g1, rowp, colp)
    return _out_call(a0, a1, W3, b3.reshape(1, MP), W_out, b_out.reshape(1, OUT))


# speedup vs baseline: 9.2359x; 1.4648x over previous
import jax
import jax.numpy as jnp
from jax import lax
from jax.experimental import pallas as pl
from jax.experimental.pallas import tpu as pltpu
from jax.experimental.pallas import tpu_sc as plsc

N = 10000
E = 320000
IN_DIM = 128
MP = 256
HALF = 128
OUT = 128

NT = 16
NC = 2
CH = 128
NCHT = 162
EPT = NCHT * CH
E_PAD = EPT * NT
NCHUNKS = E_PAD // CH
NBUF = 3
NWB = 79


def _sc_mesh():
    return plsc.VectorSubcoreMesh(core_axis_name="c", subcore_axis_name="s")


def _deg_body(col2_hbm, d0_hbm, d1_hbm, acc_sh, ones_v, idx_v, *sems):
    c = lax.axis_index("c")
    s = lax.axis_index("s")

    def fill_ones(i, _):
        ones_v[pl.ds(i * 16, 16)] = jnp.ones((16,), jnp.float32)
        return 0
    lax.fori_loop(0, 64, fill_ones, 0)

    @pl.when(s < 10)
    def _():
        pltpu.sync_copy(ones_v.at[pl.ds(0, 1000)], acc_sh.at[pl.ds(s * 1000, 1000)])

    @pl.when(s == 10)
    def _():
        pltpu.sync_copy(ones_v.at[pl.ds(0, 8)], acc_sh.at[pl.ds(N, 8)])

    plsc.subcore_barrier()

    nch = NCHT // NC

    def outer(j, _):
        d_idx = []
        for b in range(NBUF):
            i = (c * NT + s) * nch + j * NBUF + b
            d_idx.append(pltpu.async_copy(col2_hbm.at[i], idx_v.at[b], sems[b]))
        d_s = []
        for b in range(NBUF):
            d_idx[b].wait()
            d_s.append(pltpu.async_copy(
                ones_v.at[pl.ds(0, CH)], acc_sh.at[idx_v.at[b, 0]],
                sems[NBUF + b], add=True))
        for b in range(NBUF):
            d_s[b].wait()
        return 0
    lax.fori_loop(0, nch // NBUF, outer, 0)

    plsc.subcore_barrier()

    @pl.when(s < 10)
    def _():
        pltpu.sync_copy(acc_sh.at[pl.ds(s * 1000, 1000)], ones_v.at[pl.ds(0, 1000)])

        @pl.when(c == 0)
        def _():
            pltpu.sync_copy(ones_v.at[pl.ds(0, 1000)], d0_hbm.at[pl.ds(s * 1000, 1000)])

        @pl.when(c == 1)
        def _():
            pltpu.sync_copy(ones_v.at[pl.ds(0, 1000)], d1_hbm.at[pl.ds(s * 1000, 1000)])


def _deg_call(col2):
    f = pl.kernel(
        _deg_body,
        out_type=[jax.ShapeDtypeStruct((N,), jnp.float32)] * 2,
        mesh=_sc_mesh(),
        scratch_types=[
            pltpu.VMEM_SHARED((N + 8,), jnp.float32),
            pltpu.VMEM((1024,), jnp.float32),
            pltpu.VMEM((NBUF, 1, CH), jnp.int32),
        ] + [pltpu.SemaphoreType.DMA] * (2 * NBUF),
    )
    return f(col2)


def _scatter_body(g0_hbm, g1_hbm, idx2_hbm, out0_hbm, out1_hbm,
                  acc_sh, idx_v, rows_v, *sems):
    c = lax.axis_index("c")
    s = lax.axis_index("s")

    for k in range(5):
        kk = k * 16 + s
        r0 = kk * CH

        @pl.when(kk < NWB - 1)
        def _(r0=r0):
            @pl.when(c == 0)
            def _():
                pltpu.sync_copy(g0_hbm.at[pl.ds(r0, CH)], rows_v.at[0])

            @pl.when(c == 1)
            def _():
                pltpu.sync_copy(g1_hbm.at[pl.ds(r0, CH)], rows_v.at[0])

            pltpu.sync_copy(rows_v.at[0], acc_sh.at[pl.ds(r0, CH)])

        @pl.when(kk == NWB - 1)
        def _():
            @pl.when(c == 0)
            def _():
                pltpu.sync_copy(g0_hbm.at[pl.ds(9984, 16)], rows_v.at[0, pl.ds(0, 16)])

            @pl.when(c == 1)
            def _():
                pltpu.sync_copy(g1_hbm.at[pl.ds(9984, 16)], rows_v.at[0, pl.ds(0, 16)])

            pltpu.sync_copy(rows_v.at[0, pl.ds(0, 16)], acc_sh.at[pl.ds(9984, 16)])

    plsc.subcore_barrier()

    def outer(j, _):
        d_idx = []
        for b in range(NBUF):
            i = s * NCHT + j * NBUF + b
            d_idx.append(pltpu.async_copy(idx2_hbm.at[i], idx_v.at[b], sems[b]))
        d_g = []
        for b in range(NBUF):
            d_idx[b].wait()

            @pl.when(c == 0)
            def _(b=b):
                d = pltpu.async_copy(g0_hbm.at[idx_v.at[b, 0]], rows_v.at[b],
                                     sems[NBUF + b])

            @pl.when(c == 1)
            def _(b=b):
                d = pltpu.async_copy(g1_hbm.at[idx_v.at[b, 0]], rows_v.at[b],
                                     sems[NBUF + b])

            d_g.append(pltpu.make_async_copy(g0_hbm.at[idx_v.at[b, 0]],
                                             rows_v.at[b], sems[NBUF + b]))
        d_s = []
        for b in range(NBUF):
            d_g[b].wait()
            d_s.append(pltpu.async_copy(rows_v.at[b], acc_sh.at[idx_v.at[b, 1]],
                                        sems[2 * NBUF + b], add=True))
        for b in range(NBUF):
            d_s[b].wait()
        return 0
    lax.fori_loop(0, NCHT // NBUF, outer, 0)

    plsc.subcore_barrier()

    for k in range(5):
        kk = k * 16 + s
        r0 = kk * CH

        @pl.when(kk < NWB - 1)
        def _(r0=r0):
            pltpu.sync_copy(acc_sh.at[pl.ds(r0, CH)], rows_v.at[0])

            @pl.when(c == 0)
            def _():
                pltpu.sync_copy(rows_v.at[0], out0_hbm.at[pl.ds(r0, CH)])

            @pl.when(c == 1)
            def _():
                pltpu.sync_copy(rows_v.at[0], out1_hbm.at[pl.ds(r0, CH)])

        @pl.when(kk == NWB - 1)
        def _():
            pltpu.sync_copy(acc_sh.at[pl.ds(9984, 16)], rows_v.at[0, pl.ds(0, 16)])

            @pl.when(c == 0)
            def _():
                pltpu.sync_copy(rows_v.at[0, pl.ds(0, 16)], out0_hbm.at[pl.ds(9984, 16)])

            @pl.when(c == 1)
            def _():
                pltpu.sync_copy(rows_v.at[0, pl.ds(0, 16)], out1_hbm.at[pl.ds(9984, 16)])


def _scatter_call(g0, g1, idx2):
    f = pl.kernel(
        _scatter_body,
        out_type=[
            jax.ShapeDtypeStruct((N, HALF), jnp.float32),
            jax.ShapeDtypeStruct((N, HALF), jnp.float32),
        ],
        mesh=_sc_mesh(),
        scratch_types=[
            pltpu.VMEM_SHARED((N + 8, HALF), jnp.float32),
            pltpu.VMEM((NBUF, 2, CH), jnp.int32),
            pltpu.VMEM((NBUF, CH, HALF), jnp.float32),
        ] + [pltpu.SemaphoreType.DMA] * (3 * NBUF),
    )
    return f(g0, g1, idx2)


BLK = 1000


def _gelu(x):
    return x * 0.5 * (1.0 + lax.erf(x * 0.7071067811865476))


def _in_body(x_ref, w_ref, b_ref, deg_ref, g0_ref, g1_ref):
    h = jnp.dot(x_ref[...], w_ref[...], preferred_element_type=jnp.float32)
    h = h + b_ref[...]
    g = _gelu(h) / deg_ref[:, 0:1]
    g0_ref[...] = g[:, :HALF]
    g1_ref[...] = g[:, HALF:]


def _mid_body(a0_ref, a1_ref, w_ref, b_ref, deg_ref, g0_ref, g1_ref):
    h = jnp.dot(a0_ref[...], w_ref[:HALF, :], preferred_element_type=jnp.float32)
    h = h + jnp.dot(a1_ref[...], w_ref[HALF:, :], preferred_element_type=jnp.float32)
    h = h + b_ref[...]
    g = _gelu(h) / deg_ref[:, 0:1]
    g0_ref[...] = g[:, :HALF]
    g1_ref[...] = g[:, HALF:]


def _out_body(a0_ref, a1_ref, w_ref, b_ref, wo_ref, bo_ref, o_ref):
    h = jnp.dot(a0_ref[...], w_ref[:HALF, :], preferred_element_type=jnp.float32)
    h = h + jnp.dot(a1_ref[...], w_ref[HALF:, :], preferred_element_type=jnp.float32)
    h = h + b_ref[...]
    o_ref[...] = jnp.dot(h, wo_ref[...], preferred_element_type=jnp.float32) + bo_ref[...]


def _block(shape):
    return pl.BlockSpec(shape, lambda i: (0,) * len(shape))


def _rows(shape):
    return pl.BlockSpec(shape, lambda i: (i,) + (0,) * (len(shape) - 1))


def _in_call(x, w, b2, deg8):
    return pl.pallas_call(
        _in_body,
        grid=(N // BLK,),
        in_specs=[_rows((BLK, IN_DIM)), _block((IN_DIM, MP)), _block((1, MP)),
                  _rows((BLK, 8))],
        out_specs=[_rows((BLK, HALF)), _rows((BLK, HALF))],
        out_shape=[jax.ShapeDtypeStruct((N, HALF), jnp.float32)] * 2,
    )(x, w, b2, deg8)


def _mid_call(a0, a1, w, b2, deg8):
    return pl.pallas_call(
        _mid_body,
        grid=(N // BLK,),
        in_specs=[_rows((BLK, HALF)), _rows((BLK, HALF)), _block((MP, MP)),
                  _block((1, MP)), _rows((BLK, 8))],
        out_specs=[_rows((BLK, HALF)), _rows((BLK, HALF))],
        out_shape=[jax.ShapeDtypeStruct((N, HALF), jnp.float32)] * 2,
    )(a0, a1, w, b2, deg8)


def _out_call(a0, a1, w, b2, wo, bo2):
    return pl.pallas_call(
        _out_body,
        grid=(N // BLK,),
        in_specs=[_rows((BLK, HALF)), _rows((BLK, HALF)), _block((MP, MP)),
                  _block((1, MP)), _block((MP, OUT)), _block((1, OUT))],
        out_specs=_rows((BLK, OUT)),
        out_shape=jax.ShapeDtypeStruct((N, OUT), jnp.float32),
    )(a0, a1, w, b2, wo, bo2)


def kernel(x, edge_index, W_in, b_in, W1, b1, W2, b2, W3, b3, W_out, b_out):
    npad = E_PAD - E
    pad_row = (jnp.arange(npad, dtype=jnp.int32) % NT) * 624
    pad_col = N + (jnp.arange(npad, dtype=jnp.int32) % 8)
    rowp = jnp.concatenate([edge_index[0], pad_row])
    colp = jnp.concatenate([edge_index[1], pad_col])
    idx2 = jnp.stack([rowp.reshape(NCHUNKS, CH), colp.reshape(NCHUNKS, CH)], axis=1)
    col2 = colp.reshape(NCHUNKS, 1, CH)

    d0, d1 = _deg_call(col2)
    deg8 = jnp.broadcast_to((d0 + d1 - 1.0)[:, None], (N, 8))

    g0, g1 = _in_call(x, W_in, b_in.reshape(1, MP), deg8)
    for Wl, bl in ((W1, b1), (W2, b2)):
        a0, a1 = _scatter_call(g0, g1, idx2)
        g0, g1 = _mid_call(a0, a1, Wl, bl.reshape(1, MP), deg8)
    a0, a1 = _scatter_call(g0, g1, idx2)
    return _out_call(a0, a1, W3, b3.reshape(1, MP), W_out, b_out.reshape(1, OUT))


# speedup vs baseline: 11.5038x; 1.2455x over previous
import jax
import jax.numpy as jnp
from jax import lax
from jax.experimental import pallas as pl
from jax.experimental.pallas import tpu as pltpu
from jax.experimental.pallas import tpu_sc as plsc

N = 10000
E = 320000
IN_DIM = 128
MP = 256
HALF = 128
OUT = 128

NT = 16
NC = 2
CH = 128
NCHT = 162
EPT = NCHT * CH
E_PAD = EPT * NT
NCHUNKS = E_PAD // CH
NBUF = 3
NIB = 6
NWB = 79


def _sc_mesh():
    return plsc.VectorSubcoreMesh(core_axis_name="c", subcore_axis_name="s")


def _deg_body(col2_hbm, d0_hbm, d1_hbm, acc_sh, ones_v, idx_v, *sems):
    c = lax.axis_index("c")
    s = lax.axis_index("s")

    def fill_ones(i, _):
        ones_v[pl.ds(i * 16, 16)] = jnp.ones((16,), jnp.float32)
        return 0
    lax.fori_loop(0, 64, fill_ones, 0)

    @pl.when(s < 10)
    def _():
        pltpu.sync_copy(ones_v.at[pl.ds(0, 1000)], acc_sh.at[pl.ds(s * 1000, 1000)])

    @pl.when(s == 10)
    def _():
        pltpu.sync_copy(ones_v.at[pl.ds(0, 8)], acc_sh.at[pl.ds(N, 8)])

    plsc.subcore_barrier()

    nch = NCHT // NC

    def outer(j, _):
        d_idx = []
        for b in range(NBUF):
            i = (c * NT + s) * nch + j * NBUF + b
            d_idx.append(pltpu.async_copy(col2_hbm.at[i], idx_v.at[b], sems[b]))
        d_s = []
        for b in range(NBUF):
            d_idx[b].wait()
            d_s.append(pltpu.async_copy(
                ones_v.at[pl.ds(0, CH)], acc_sh.at[idx_v.at[b, 0]],
                sems[NBUF + b], add=True))
        for b in range(NBUF):
            d_s[b].wait()
        return 0
    lax.fori_loop(0, nch // NBUF, outer, 0)

    plsc.subcore_barrier()

    @pl.when(s < 10)
    def _():
        pltpu.sync_copy(acc_sh.at[pl.ds(s * 1000, 1000)], ones_v.at[pl.ds(0, 1000)])

        @pl.when(c == 0)
        def _():
            pltpu.sync_copy(ones_v.at[pl.ds(0, 1000)], d0_hbm.at[pl.ds(s * 1000, 1000)])

        @pl.when(c == 1)
        def _():
            pltpu.sync_copy(ones_v.at[pl.ds(0, 1000)], d1_hbm.at[pl.ds(s * 1000, 1000)])


def _deg_call(col2):
    f = pl.kernel(
        _deg_body,
        out_type=[jax.ShapeDtypeStruct((N,), jnp.float32)] * 2,
        mesh=_sc_mesh(),
        scratch_types=[
            pltpu.VMEM_SHARED((N + 8,), jnp.float32),
            pltpu.VMEM((1024,), jnp.float32),
            pltpu.VMEM((NBUF, 1, CH), jnp.int32),
        ] + [pltpu.SemaphoreType.DMA] * (2 * NBUF),
    )
    return f(col2)


def _scatter_body(g0_hbm, g1_hbm, idx2_hbm, out0_hbm, out1_hbm,
                  acc_sh, idx_v, rows_v, *sems):
    c = lax.axis_index("c")
    s = lax.axis_index("s")

    for k in range(5):
        kk = k * 16 + s
        r0 = kk * CH

        @pl.when(kk < NWB - 1)
        def _(r0=r0):
            @pl.when(c == 0)
            def _():
                pltpu.sync_copy(g0_hbm.at[pl.ds(r0, CH)], rows_v.at[0])

            @pl.when(c == 1)
            def _():
                pltpu.sync_copy(g1_hbm.at[pl.ds(r0, CH)], rows_v.at[0])

            pltpu.sync_copy(rows_v.at[0], acc_sh.at[pl.ds(r0, CH)])

        @pl.when(kk == NWB - 1)
        def _():
            @pl.when(c == 0)
            def _():
                pltpu.sync_copy(g0_hbm.at[pl.ds(9984, 16)], rows_v.at[0, pl.ds(0, 16)])

            @pl.when(c == 1)
            def _():
                pltpu.sync_copy(g1_hbm.at[pl.ds(9984, 16)], rows_v.at[0, pl.ds(0, 16)])

            pltpu.sync_copy(rows_v.at[0, pl.ds(0, 16)], acc_sh.at[pl.ds(9984, 16)])

    plsc.subcore_barrier()

    sem_i = sems[:NIB]
    sem_g = sems[NIB:NIB + NBUF]
    sem_s = sems[NIB + NBUF:]
    base = s * NCHT

    for bi in range(NIB):
        pltpu.async_copy(idx2_hbm.at[base + bi], idx_v.at[bi], sem_i[bi])

    def _gather(bi, b, i):
        @pl.when(c == 0)
        def _():
            pltpu.async_copy(g0_hbm.at[idx_v.at[bi, 0]], rows_v.at[b], sem_g[b])

        @pl.when(c == 1)
        def _():
            pltpu.async_copy(g1_hbm.at[idx_v.at[bi, 0]], rows_v.at[b], sem_g[b])

        return pltpu.make_async_copy(g0_hbm.at[idx_v.at[bi, 0]], rows_v.at[b],
                                     sem_g[b])

    def outer(j, _):
        for bi in range(NIB):
            b = bi % NBUF
            i = base + j * NIB + bi
            pi = (bi - NBUF) % NIB

            def drain_and_prefetch(b=b, bi=bi, pi=pi, i=i):
                pltpu.make_async_copy(rows_v.at[b], acc_sh.at[idx_v.at[bi, 1]],
                                      sem_s[b]).wait()

                @pl.when(i + NBUF < base + NCHT)
                def _():
                    pltpu.async_copy(idx2_hbm.at[i + NBUF], idx_v.at[pi], sem_i[pi])

            if bi >= NBUF:
                drain_and_prefetch()
            else:
                @pl.when(j > 0)
                def _(fn=drain_and_prefetch):
                    fn()

            pltpu.make_async_copy(idx2_hbm.at[i], idx_v.at[bi], sem_i[bi]).wait()
            _gather(bi, b, i).wait()
            pltpu.async_copy(rows_v.at[b], acc_sh.at[idx_v.at[bi, 1]],
                             sem_s[b], add=True)

        return 0
    lax.fori_loop(0, NCHT // NIB, outer, 0)

    for b in range(NBUF):
        bi = NIB - NBUF + b
        pltpu.make_async_copy(rows_v.at[b], acc_sh.at[idx_v.at[bi, 1]],
                              sem_s[b]).wait()

    plsc.subcore_barrier()

    for k in range(5):
        kk = k * 16 + s
        r0 = kk * CH

        @pl.when(kk < NWB - 1)
        def _(r0=r0):
            pltpu.sync_copy(acc_sh.at[pl.ds(r0, CH)], rows_v.at[0])

            @pl.when(c == 0)
            def _():
                pltpu.sync_copy(rows_v.at[0], out0_hbm.at[pl.ds(r0, CH)])

            @pl.when(c == 1)
            def _():
                pltpu.sync_copy(rows_v.at[0], out1_hbm.at[pl.ds(r0, CH)])

        @pl.when(kk == NWB - 1)
        def _():
            pltpu.sync_copy(acc_sh.at[pl.ds(9984, 16)], rows_v.at[0, pl.ds(0, 16)])

            @pl.when(c == 0)
            def _():
                pltpu.sync_copy(rows_v.at[0, pl.ds(0, 16)], out0_hbm.at[pl.ds(9984, 16)])

            @pl.when(c == 1)
            def _():
                pltpu.sync_copy(rows_v.at[0, pl.ds(0, 16)], out1_hbm.at[pl.ds(9984, 16)])


def _scatter_call(g0, g1, idx2):
    f = pl.kernel(
        _scatter_body,
        out_type=[
            jax.ShapeDtypeStruct((N, HALF), jnp.float32),
            jax.ShapeDtypeStruct((N, HALF), jnp.float32),
        ],
        mesh=_sc_mesh(),
        scratch_types=[
            pltpu.VMEM_SHARED((N + 8, HALF), jnp.float32),
            pltpu.VMEM((NIB, 2, CH), jnp.int32),
            pltpu.VMEM((NBUF, CH, HALF), jnp.float32),
        ] + [pltpu.SemaphoreType.DMA] * (NIB + 2 * NBUF),
    )
    return f(g0, g1, idx2)


BLK = 1000


def _gelu(x):
    return x * 0.5 * (1.0 + lax.erf(x * 0.7071067811865476))


def _in_body(x_ref, w_ref, b_ref, deg_ref, g0_ref, g1_ref):
    h = jnp.dot(x_ref[...], w_ref[...], preferred_element_type=jnp.float32)
    h = h + b_ref[...]
    g = _gelu(h) / deg_ref[:, 0:1]
    g0_ref[...] = g[:, :HALF]
    g1_ref[...] = g[:, HALF:]


def _mid_body(a0_ref, a1_ref, w_ref, b_ref, deg_ref, g0_ref, g1_ref):
    h = jnp.dot(a0_ref[...], w_ref[:HALF, :], preferred_element_type=jnp.float32)
    h = h + jnp.dot(a1_ref[...], w_ref[HALF:, :], preferred_element_type=jnp.float32)
    h = h + b_ref[...]
    g = _gelu(h) / deg_ref[:, 0:1]
    g0_ref[...] = g[:, :HALF]
    g1_ref[...] = g[:, HALF:]


def _out_body(a0_ref, a1_ref, w_ref, b_ref, wo_ref, bo_ref, o_ref):
    h = jnp.dot(a0_ref[...], w_ref[:HALF, :], preferred_element_type=jnp.float32)
    h = h + jnp.dot(a1_ref[...], w_ref[HALF:, :], preferred_element_type=jnp.float32)
    h = h + b_ref[...]
    o_ref[...] = jnp.dot(h, wo_ref[...], preferred_element_type=jnp.float32) + bo_ref[...]


def _block(shape):
    return pl.BlockSpec(shape, lambda i: (0,) * len(shape))


def _rows(shape):
    return pl.BlockSpec(shape, lambda i: (i,) + (0,) * (len(shape) - 1))


def _in_call(x, w, b2, deg8):
    return pl.pallas_call(
        _in_body,
        grid=(N // BLK,),
        in_specs=[_rows((BLK, IN_DIM)), _block((IN_DIM, MP)), _block((1, MP)),
                  _rows((BLK, 8))],
        out_specs=[_rows((BLK, HALF)), _rows((BLK, HALF))],
        out_shape=[jax.ShapeDtypeStruct((N, HALF), jnp.float32)] * 2,
    )(x, w, b2, deg8)


def _mid_call(a0, a1, w, b2, deg8):
    return pl.pallas_call(
        _mid_body,
        grid=(N // BLK,),
        in_specs=[_rows((BLK, HALF)), _rows((BLK, HALF)), _block((MP, MP)),
                  _block((1, MP)), _rows((BLK, 8))],
        out_specs=[_rows((BLK, HALF)), _rows((BLK, HALF))],
        out_shape=[jax.ShapeDtypeStruct((N, HALF), jnp.float32)] * 2,
    )(a0, a1, w, b2, deg8)


def _out_call(a0, a1, w, b2, wo, bo2):
    return pl.pallas_call(
        _out_body,
        grid=(N // BLK,),
        in_specs=[_rows((BLK, HALF)), _rows((BLK, HALF)), _block((MP, MP)),
                  _block((1, MP)), _block((MP, OUT)), _block((1, OUT))],
        out_specs=_rows((BLK, OUT)),
        out_shape=jax.ShapeDtypeStruct((N, OUT), jnp.float32),
    )(a0, a1, w, b2, wo, bo2)


def kernel(x, edge_index, W_in, b_in, W1, b1, W2, b2, W3, b3, W_out, b_out):
    npad = E_PAD - E
    pad_row = (jnp.arange(npad, dtype=jnp.int32) % NT) * 624
    pad_col = N + (jnp.arange(npad, dtype=jnp.int32) % 8)
    rowp = jnp.concatenate([edge_index[0], pad_row])
    colp = jnp.concatenate([edge_index[1], pad_col])
    idx2 = jnp.stack([rowp.reshape(NCHUNKS, CH), colp.reshape(NCHUNKS, CH)], axis=1)
    col2 = colp.reshape(NCHUNKS, 1, CH)

    d0, d1 = _deg_call(col2)
    deg8 = jnp.broadcast_to((d0 + d1 - 1.0)[:, None], (N, 8))

    g0, g1 = _in_call(x, W_in, b_in.reshape(1, MP), deg8)
    for Wl, bl in ((W1, b1), (W2, b2)):
        a0, a1 = _scatter_call(g0, g1, idx2)
        g0, g1 = _mid_call(a0, a1, Wl, bl.reshape(1, MP), deg8)
    a0, a1 = _scatter_call(g0, g1, idx2)
    return _out_call(a0, a1, W3, b3.reshape(1, MP), W_out, b_out.reshape(1, OUT))


# speedup vs baseline: 14.4265x; 1.2541x over previous
import jax
import jax.numpy as jnp
from jax import lax
from jax.experimental import pallas as pl
from jax.experimental.pallas import tpu as pltpu
from jax.experimental.pallas import tpu_sc as plsc

N = 10000
E = 320000
IN_DIM = 128
MP = 256
HALF = 128
OUT = 128

NT = 16
NC = 2
CH = 128
NCHT = 162
EPT = NCHT * CH
E_PAD = EPT * NT
NCHUNKS = E_PAD // CH
NBUF = 3
NIB = 6
NWB = 79


def _sc_mesh():
    return plsc.VectorSubcoreMesh(core_axis_name="c", subcore_axis_name="s")


def _deg_body(col2_hbm, d0_hbm, d1_hbm, acc_sh, ones_v, idx_v, *sems):
    c = lax.axis_index("c")
    s = lax.axis_index("s")

    def fill_ones(i, _):
        ones_v[pl.ds(i * 16, 16)] = jnp.ones((16,), jnp.float32)
        return 0
    lax.fori_loop(0, 64, fill_ones, 0)

    @pl.when(s < 10)
    def _():
        pltpu.sync_copy(ones_v.at[pl.ds(0, 1000)], acc_sh.at[pl.ds(s * 1000, 1000)])

    @pl.when(s == 10)
    def _():
        pltpu.sync_copy(ones_v.at[pl.ds(0, 8)], acc_sh.at[pl.ds(N, 8)])

    plsc.subcore_barrier()

    nch = NCHT // NC

    def outer(j, _):
        d_idx = []
        for b in range(NBUF):
            i = (c * NT + s) * nch + j * NBUF + b
            d_idx.append(pltpu.async_copy(col2_hbm.at[i], idx_v.at[b], sems[b]))
        d_s = []
        for b in range(NBUF):
            d_idx[b].wait()
            d_s.append(pltpu.async_copy(
                ones_v.at[pl.ds(0, CH)], acc_sh.at[idx_v.at[b, 0]],
                sems[NBUF + b], add=True))
        for b in range(NBUF):
            d_s[b].wait()
        return 0
    lax.fori_loop(0, nch // NBUF, outer, 0)

    plsc.subcore_barrier()

    @pl.when(s < 10)
    def _():
        pltpu.sync_copy(acc_sh.at[pl.ds(s * 1000, 1000)], ones_v.at[pl.ds(0, 1000)])

        @pl.when(c == 0)
        def _():
            pltpu.sync_copy(ones_v.at[pl.ds(0, 1000)], d0_hbm.at[pl.ds(s * 1000, 1000)])

        @pl.when(c == 1)
        def _():
            pltpu.sync_copy(ones_v.at[pl.ds(0, 1000)], d1_hbm.at[pl.ds(s * 1000, 1000)])


def _deg_call(col2):
    f = pl.kernel(
        _deg_body,
        out_type=[jax.ShapeDtypeStruct((N,), jnp.float32)] * 2,
        mesh=_sc_mesh(),
        scratch_types=[
            pltpu.VMEM_SHARED((N + 8,), jnp.float32),
            pltpu.VMEM((1024,), jnp.float32),
            pltpu.VMEM((NBUF, 1, CH), jnp.int32),
        ] + [pltpu.SemaphoreType.DMA] * (2 * NBUF),
    )
    return f(col2)


def _scatter_body(g0_hbm, g1_hbm, idx2_hbm, out0_hbm, out1_hbm,
                  acc_sh, idx_v, rows_v, *sems):
    c = lax.axis_index("c")
    s = lax.axis_index("s")

    for k in range(5):
        kk = k * 16 + s
        r0 = kk * CH

        @pl.when(kk < NWB - 1)
        def _(r0=r0):
            @pl.when(c == 0)
            def _():
                pltpu.sync_copy(g0_hbm.at[pl.ds(r0, CH)], rows_v.at[0])

            @pl.when(c == 1)
            def _():
                pltpu.sync_copy(g1_hbm.at[pl.ds(r0, CH)], rows_v.at[0])

            pltpu.sync_copy(rows_v.at[0], acc_sh.at[pl.ds(r0, CH)])

        @pl.when(kk == NWB - 1)
        def _():
            @pl.when(c == 0)
            def _():
                pltpu.sync_copy(g0_hbm.at[pl.ds(9984, 16)], rows_v.at[0, pl.ds(0, 16)])

            @pl.when(c == 1)
            def _():
                pltpu.sync_copy(g1_hbm.at[pl.ds(9984, 16)], rows_v.at[0, pl.ds(0, 16)])

            pltpu.sync_copy(rows_v.at[0, pl.ds(0, 16)], acc_sh.at[pl.ds(9984, 16)])

    plsc.subcore_barrier()

    sem_i = sems[:NIB]
    sem_g = sems[NIB:NIB + NBUF]
    sem_s = sems[NIB + NBUF:]
    base = s * NCHT

    for bi in range(NIB):
        pltpu.async_copy(idx2_hbm.at[base + bi], idx_v.at[bi], sem_i[bi])

    def _gather(bi, b, i):
        @pl.when(c == 0)
        def _():
            pltpu.async_copy(g0_hbm.at[idx_v.at[bi, 0]], rows_v.at[b], sem_g[b])

        @pl.when(c == 1)
        def _():
            pltpu.async_copy(g1_hbm.at[idx_v.at[bi, 0]], rows_v.at[b], sem_g[b])

        return pltpu.make_async_copy(g0_hbm.at[idx_v.at[bi, 0]], rows_v.at[b],
                                     sem_g[b])

    def outer(j, _):
        for bi in range(NIB):
            b = bi % NBUF
            i = base + j * NIB + bi
            pi = (bi - NBUF) % NIB

            def drain_and_prefetch(b=b, bi=bi, pi=pi, i=i):
                pltpu.make_async_copy(rows_v.at[b], acc_sh.at[idx_v.at[bi, 1]],
                                      sem_s[b]).wait()

                @pl.when(i + NBUF < base + NCHT)
                def _():
                    pltpu.async_copy(idx2_hbm.at[i + NBUF], idx_v.at[pi], sem_i[pi])

            if bi >= NBUF:
                drain_and_prefetch()
            else:
                @pl.when(j > 0)
                def _(fn=drain_and_prefetch):
                    fn()

            pltpu.make_async_copy(idx2_hbm.at[i], idx_v.at[bi], sem_i[bi]).wait()
            _gather(bi, b, i)

            pb, pbuf = (bi - 1) % NIB, (b - 1) % NBUF

            def prev_scatter(pb=pb, pbuf=pbuf):
                pltpu.make_async_copy(g0_hbm.at[idx_v.at[pb, 0]],
                                      rows_v.at[pbuf], sem_g[pbuf]).wait()
                pltpu.async_copy(rows_v.at[pbuf], acc_sh.at[idx_v.at[pb, 1]],
                                 sem_s[pbuf], add=True)

            if bi >= 1:
                prev_scatter()
            else:
                @pl.when(j > 0)
                def _(fn=prev_scatter):
                    fn()

        return 0
    lax.fori_loop(0, NCHT // NIB, outer, 0)

    pltpu.make_async_copy(g0_hbm.at[idx_v.at[NIB - 1, 0]],
                          rows_v.at[NBUF - 1], sem_g[NBUF - 1]).wait()
    pltpu.async_copy(rows_v.at[NBUF - 1], acc_sh.at[idx_v.at[NIB - 1, 1]],
                     sem_s[NBUF - 1], add=True)
    for b in range(NBUF):
        bi = NIB - NBUF + b
        pltpu.make_async_copy(rows_v.at[b], acc_sh.at[idx_v.at[bi, 1]],
                              sem_s[b]).wait()

    plsc.subcore_barrier()

    for k in range(5):
        kk = k * 16 + s
        r0 = kk * CH

        @pl.when(kk < NWB - 1)
        def _(r0=r0):
            pltpu.sync_copy(acc_sh.at[pl.ds(r0, CH)], rows_v.at[0])

            @pl.when(c == 0)
            def _():
                pltpu.sync_copy(rows_v.at[0], out0_hbm.at[pl.ds(r0, CH)])

            @pl.when(c == 1)
            def _():
                pltpu.sync_copy(rows_v.at[0], out1_hbm.at[pl.ds(r0, CH)])

        @pl.when(kk == NWB - 1)
        def _():
            pltpu.sync_copy(acc_sh.at[pl.ds(9984, 16)], rows_v.at[0, pl.ds(0, 16)])

            @pl.when(c == 0)
            def _():
                pltpu.sync_copy(rows_v.at[0, pl.ds(0, 16)], out0_hbm.at[pl.ds(9984, 16)])

            @pl.when(c == 1)
            def _():
                pltpu.sync_copy(rows_v.at[0, pl.ds(0, 16)], out1_hbm.at[pl.ds(9984, 16)])


def _scatter_call(g0, g1, idx2):
    f = pl.kernel(
        _scatter_body,
        out_type=[
            jax.ShapeDtypeStruct((N, HALF), jnp.float32),
            jax.ShapeDtypeStruct((N, HALF), jnp.float32),
        ],
        mesh=_sc_mesh(),
        scratch_types=[
            pltpu.VMEM_SHARED((N + 8, HALF), jnp.float32),
            pltpu.VMEM((NIB, 2, CH), jnp.int32),
            pltpu.VMEM((NBUF, CH, HALF), jnp.float32),
        ] + [pltpu.SemaphoreType.DMA] * (NIB + 2 * NBUF),
    )
    return f(g0, g1, idx2)


BLK = 1000


def _gelu(x):
    return x * 0.5 * (1.0 + lax.erf(x * 0.7071067811865476))


def _in_body(x_ref, w_ref, b_ref, deg_ref, g0_ref, g1_ref):
    h = jnp.dot(x_ref[...], w_ref[...], preferred_element_type=jnp.float32)
    h = h + b_ref[...]
    g = _gelu(h) / deg_ref[:, 0:1]
    g0_ref[...] = g[:, :HALF]
    g1_ref[...] = g[:, HALF:]


def _mid_body(a0_ref, a1_ref, w_ref, b_ref, deg_ref, g0_ref, g1_ref):
    h = jnp.dot(a0_ref[...], w_ref[:HALF, :], preferred_element_type=jnp.float32)
    h = h + jnp.dot(a1_ref[...], w_ref[HALF:, :], preferred_element_type=jnp.float32)
    h = h + b_ref[...]
    g = _gelu(h) / deg_ref[:, 0:1]
    g0_ref[...] = g[:, :HALF]
    g1_ref[...] = g[:, HALF:]


def _out_body(a0_ref, a1_ref, w_ref, b_ref, wo_ref, bo_ref, o_ref):
    h = jnp.dot(a0_ref[...], w_ref[:HALF, :], preferred_element_type=jnp.float32)
    h = h + jnp.dot(a1_ref[...], w_ref[HALF:, :], preferred_element_type=jnp.float32)
    h = h + b_ref[...]
    o_ref[...] = jnp.dot(h, wo_ref[...], preferred_element_type=jnp.float32) + bo_ref[...]


def _block(shape):
    return pl.BlockSpec(shape, lambda i: (0,) * len(shape))


def _rows(shape):
    return pl.BlockSpec(shape, lambda i: (i,) + (0,) * (len(shape) - 1))


def _in_call(x, w, b2, deg8):
    return pl.pallas_call(
        _in_body,
        grid=(N // BLK,),
        in_specs=[_rows((BLK, IN_DIM)), _block((IN_DIM, MP)), _block((1, MP)),
                  _rows((BLK, 8))],
        out_specs=[_rows((BLK, HALF)), _rows((BLK, HALF))],
        out_shape=[jax.ShapeDtypeStruct((N, HALF), jnp.float32)] * 2,
    )(x, w, b2, deg8)


def _mid_call(a0, a1, w, b2, deg8):
    return pl.pallas_call(
        _mid_body,
        grid=(N // BLK,),
        in_specs=[_rows((BLK, HALF)), _rows((BLK, HALF)), _block((MP, MP)),
                  _block((1, MP)), _rows((BLK, 8))],
        out_specs=[_rows((BLK, HALF)), _rows((BLK, HALF))],
        out_shape=[jax.ShapeDtypeStruct((N, HALF), jnp.float32)] * 2,
    )(a0, a1, w, b2, deg8)


def _out_call(a0, a1, w, b2, wo, bo2):
    return pl.pallas_call(
        _out_body,
        grid=(N // BLK,),
        in_specs=[_rows((BLK, HALF)), _rows((BLK, HALF)), _block((MP, MP)),
                  _block((1, MP)), _block((MP, OUT)), _block((1, OUT))],
        out_specs=_rows((BLK, OUT)),
        out_shape=jax.ShapeDtypeStruct((N, OUT), jnp.float32),
    )(a0, a1, w, b2, wo, bo2)


def kernel(x, edge_index, W_in, b_in, W1, b1, W2, b2, W3, b3, W_out, b_out):
    npad = E_PAD - E
    pad_row = (jnp.arange(npad, dtype=jnp.int32) % NT) * 624
    pad_col = N + (jnp.arange(npad, dtype=jnp.int32) % 8)
    rowp = jnp.concatenate([edge_index[0], pad_row])
    colp = jnp.concatenate([edge_index[1], pad_col])
    idx2 = jnp.stack([rowp.reshape(NCHUNKS, CH), colp.reshape(NCHUNKS, CH)], axis=1)
    col2 = colp.reshape(NCHUNKS, 1, CH)

    d0, d1 = _deg_call(col2)
    deg8 = jnp.broadcast_to((d0 + d1 - 1.0)[:, None], (N, 8))

    g0, g1 = _in_call(x, W_in, b_in.reshape(1, MP), deg8)
    for Wl, bl in ((W1, b1), (W2, b2)):
        a0, a1 = _scatter_call(g0, g1, idx2)
        g0, g1 = _mid_call(a0, a1, Wl, bl.reshape(1, MP), deg8)
    a0, a1 = _scatter_call(g0, g1, idx2)
    return _out_call(a0, a1, W3, b3.reshape(1, MP), W_out, b_out.reshape(1, OUT))


# speedup vs baseline: 14.6431x; 1.0150x over previous
import jax
import jax.numpy as jnp
from jax import lax
from jax.experimental import pallas as pl
from jax.experimental.pallas import tpu as pltpu
from jax.experimental.pallas import tpu_sc as plsc

N = 10000
E = 320000
IN_DIM = 128
MP = 256
HALF = 128
OUT = 128

NT = 16
NC = 2
CH = 128
NCHT = 162
EPT = NCHT * CH
E_PAD = EPT * NT
NCHUNKS = E_PAD // CH
NBUF = 3
NIB = 6
NWB = 79


def _sc_mesh():
    return plsc.VectorSubcoreMesh(core_axis_name="c", subcore_axis_name="s")


def _deg_body(col2_hbm, d0_hbm, d1_hbm, acc_sh, ones_v, idx_v, *sems):
    c = lax.axis_index("c")
    s = lax.axis_index("s")

    def fill_ones(i, _):
        ones_v[pl.ds(i * 16, 16)] = jnp.ones((16,), jnp.float32)
        return 0
    lax.fori_loop(0, 64, fill_ones, 0)

    @pl.when(s < 10)
    def _():
        pltpu.sync_copy(ones_v.at[pl.ds(0, 1000)], acc_sh.at[pl.ds(s * 1000, 1000)])

    @pl.when(s == 10)
    def _():
        pltpu.sync_copy(ones_v.at[pl.ds(0, 8)], acc_sh.at[pl.ds(N, 8)])

    plsc.subcore_barrier()

    nch = NCHT // NC

    def outer(j, _):
        d_idx = []
        for b in range(NBUF):
            i = (c * NT + s) * nch + j * NBUF + b
            d_idx.append(pltpu.async_copy(col2_hbm.at[i], idx_v.at[b], sems[b]))
        d_s = []
        for b in range(NBUF):
            d_idx[b].wait()
            d_s.append(pltpu.async_copy(
                ones_v.at[pl.ds(0, CH)], acc_sh.at[idx_v.at[b, 0]],
                sems[NBUF + b], add=True))
        for b in range(NBUF):
            d_s[b].wait()
        return 0
    lax.fori_loop(0, nch // NBUF, outer, 0)

    plsc.subcore_barrier()

    @pl.when(s < 10)
    def _():
        pltpu.sync_copy(acc_sh.at[pl.ds(s * 1000, 1000)], ones_v.at[pl.ds(0, 1000)])

        @pl.when(c == 0)
        def _():
            pltpu.sync_copy(ones_v.at[pl.ds(0, 1000)], d0_hbm.at[pl.ds(s * 1000, 1000)])

        @pl.when(c == 1)
        def _():
            pltpu.sync_copy(ones_v.at[pl.ds(0, 1000)], d1_hbm.at[pl.ds(s * 1000, 1000)])


def _deg_call(col2):
    f = pl.kernel(
        _deg_body,
        out_type=[jax.ShapeDtypeStruct((N,), jnp.float32)] * 2,
        mesh=_sc_mesh(),
        scratch_types=[
            pltpu.VMEM_SHARED((N + 8,), jnp.float32),
            pltpu.VMEM((1024,), jnp.float32),
            pltpu.VMEM((NBUF, 1, CH), jnp.int32),
        ] + [pltpu.SemaphoreType.DMA] * (2 * NBUF),
    )
    return f(col2)


def _scatter_body(g0_hbm, g1_hbm, idx2_hbm, out0_hbm, out1_hbm,
                  acc_sh, idx_v, rows_v, *sems):
    c = lax.axis_index("c")
    s = lax.axis_index("s")

    for k in range(5):
        kk = k * 16 + s
        r0 = kk * CH

        @pl.when(kk < NWB - 1)
        def _(r0=r0):
            @pl.when(c == 0)
            def _():
                pltpu.sync_copy(g0_hbm.at[pl.ds(r0, CH)], acc_sh.at[pl.ds(r0, CH)])

            @pl.when(c == 1)
            def _():
                pltpu.sync_copy(g1_hbm.at[pl.ds(r0, CH)], acc_sh.at[pl.ds(r0, CH)])

        @pl.when(kk == NWB - 1)
        def _():
            @pl.when(c == 0)
            def _():
                pltpu.sync_copy(g0_hbm.at[pl.ds(9984, 16)], acc_sh.at[pl.ds(9984, 16)])

            @pl.when(c == 1)
            def _():
                pltpu.sync_copy(g1_hbm.at[pl.ds(9984, 16)], acc_sh.at[pl.ds(9984, 16)])

    plsc.subcore_barrier()

    sem_i = sems[:NIB]
    sem_g = sems[NIB:NIB + NBUF]
    sem_s = sems[NIB + NBUF:]
    base = s * NCHT

    for bi in range(NIB):
        pltpu.async_copy(idx2_hbm.at[base + bi], idx_v.at[bi], sem_i[bi])

    def _gather(bi, b, i):
        @pl.when(c == 0)
        def _():
            pltpu.async_copy(g0_hbm.at[idx_v.at[bi, 0]], rows_v.at[b], sem_g[b])

        @pl.when(c == 1)
        def _():
            pltpu.async_copy(g1_hbm.at[idx_v.at[bi, 0]], rows_v.at[b], sem_g[b])

        return pltpu.make_async_copy(g0_hbm.at[idx_v.at[bi, 0]], rows_v.at[b],
                                     sem_g[b])

    def outer(j, _):
        for bi in range(NIB):
            b = bi % NBUF
            i = base + j * NIB + bi
            pi = (bi - NBUF) % NIB

            def drain_and_prefetch(b=b, bi=bi, pi=pi, i=i):
                pltpu.make_async_copy(rows_v.at[b], acc_sh.at[idx_v.at[bi, 1]],
                                      sem_s[b]).wait()

                @pl.when(i + NBUF < base + NCHT)
                def _():
                    pltpu.async_copy(idx2_hbm.at[i + NBUF], idx_v.at[pi], sem_i[pi])

            if bi >= NBUF:
                drain_and_prefetch()
            else:
                @pl.when(j > 0)
                def _(fn=drain_and_prefetch):
                    fn()

            pltpu.make_async_copy(idx2_hbm.at[i], idx_v.at[bi], sem_i[bi]).wait()
            _gather(bi, b, i)

            pb, pbuf = (bi - 1) % NIB, (b - 1) % NBUF

            def prev_scatter(pb=pb, pbuf=pbuf):
                pltpu.make_async_copy(g0_hbm.at[idx_v.at[pb, 0]],
                                      rows_v.at[pbuf], sem_g[pbuf]).wait()
                pltpu.async_copy(rows_v.at[pbuf], acc_sh.at[idx_v.at[pb, 1]],
                                 sem_s[pbuf], add=True)

            if bi >= 1:
                prev_scatter()
            else:
                @pl.when(j > 0)
                def _(fn=prev_scatter):
                    fn()

        return 0
    lax.fori_loop(0, NCHT // NIB, outer, 0)

    pltpu.make_async_copy(g0_hbm.at[idx_v.at[NIB - 1, 0]],
                          rows_v.at[NBUF - 1], sem_g[NBUF - 1]).wait()
    pltpu.async_copy(rows_v.at[NBUF - 1], acc_sh.at[idx_v.at[NIB - 1, 1]],
                     sem_s[NBUF - 1], add=True)
    for b in range(NBUF):
        bi = NIB - NBUF + b
        pltpu.make_async_copy(rows_v.at[b], acc_sh.at[idx_v.at[bi, 1]],
                              sem_s[b]).wait()

    plsc.subcore_barrier()

    for k in range(5):
        kk = k * 16 + s
        r0 = kk * CH

        @pl.when(kk < NWB - 1)
        def _(r0=r0):
            @pl.when(c == 0)
            def _():
                pltpu.sync_copy(acc_sh.at[pl.ds(r0, CH)], out0_hbm.at[pl.ds(r0, CH)])

            @pl.when(c == 1)
            def _():
                pltpu.sync_copy(acc_sh.at[pl.ds(r0, CH)], out1_hbm.at[pl.ds(r0, CH)])

        @pl.when(kk == NWB - 1)
        def _():
            @pl.when(c == 0)
            def _():
                pltpu.sync_copy(acc_sh.at[pl.ds(9984, 16)], out0_hbm.at[pl.ds(9984, 16)])

            @pl.when(c == 1)
            def _():
                pltpu.sync_copy(acc_sh.at[pl.ds(9984, 16)], out1_hbm.at[pl.ds(9984, 16)])


def _scatter_call(g0, g1, idx2):
    f = pl.kernel(
        _scatter_body,
        out_type=[
            jax.ShapeDtypeStruct((N, HALF), jnp.float32),
            jax.ShapeDtypeStruct((N, HALF), jnp.float32),
        ],
        mesh=_sc_mesh(),
        scratch_types=[
            pltpu.VMEM_SHARED((N + 8, HALF), jnp.float32),
            pltpu.VMEM((NIB, 2, CH), jnp.int32),
            pltpu.VMEM((NBUF, CH, HALF), jnp.float32),
        ] + [pltpu.SemaphoreType.DMA] * (NIB + 2 * NBUF),
    )
    return f(g0, g1, idx2)


BLK = 1000


def _gelu(x):
    return x * 0.5 * (1.0 + lax.erf(x * 0.7071067811865476))


def _in_body(x_ref, w_ref, b_ref, deg_ref, g0_ref, g1_ref):
    h = jnp.dot(x_ref[...], w_ref[...], preferred_element_type=jnp.float32)
    h = h + b_ref[...]
    g = _gelu(h) / deg_ref[:, 0:1]
    g0_ref[...] = g[:, :HALF]
    g1_ref[...] = g[:, HALF:]


def _mid_body(a0_ref, a1_ref, w_ref, b_ref, deg_ref, g0_ref, g1_ref):
    h = jnp.dot(a0_ref[...], w_ref[:HALF, :], preferred_element_type=jnp.float32)
    h = h + jnp.dot(a1_ref[...], w_ref[HALF:, :], preferred_element_type=jnp.float32)
    h = h + b_ref[...]
    g = _gelu(h) / deg_ref[:, 0:1]
    g0_ref[...] = g[:, :HALF]
    g1_ref[...] = g[:, HALF:]


def _out_body(a0_ref, a1_ref, w_ref, b_ref, wo_ref, bo_ref, o_ref):
    h = jnp.dot(a0_ref[...], w_ref[:HALF, :], preferred_element_type=jnp.float32)
    h = h + jnp.dot(a1_ref[...], w_ref[HALF:, :], preferred_element_type=jnp.float32)
    h = h + b_ref[...]
    o_ref[...] = jnp.dot(h, wo_ref[...], preferred_element_type=jnp.float32) + bo_ref[...]


def _block(shape):
    return pl.BlockSpec(shape, lambda i: (0,) * len(shape))


def _rows(shape):
    return pl.BlockSpec(shape, lambda i: (i,) + (0,) * (len(shape) - 1))


def _in_call(x, w, b2, deg8):
    return pl.pallas_call(
        _in_body,
        grid=(N // BLK,),
        in_specs=[_rows((BLK, IN_DIM)), _block((IN_DIM, MP)), _block((1, MP)),
                  _rows((BLK, 8))],
        out_specs=[_rows((BLK, HALF)), _rows((BLK, HALF))],
        out_shape=[jax.ShapeDtypeStruct((N, HALF), jnp.float32)] * 2,
    )(x, w, b2, deg8)


def _mid_call(a0, a1, w, b2, deg8):
    return pl.pallas_call(
        _mid_body,
        grid=(N // BLK,),
        in_specs=[_rows((BLK, HALF)), _rows((BLK, HALF)), _block((MP, MP)),
                  _block((1, MP)), _rows((BLK, 8))],
        out_specs=[_rows((BLK, HALF)), _rows((BLK, HALF))],
        out_shape=[jax.ShapeDtypeStruct((N, HALF), jnp.float32)] * 2,
    )(a0, a1, w, b2, deg8)


def _out_call(a0, a1, w, b2, wo, bo2):
    return pl.pallas_call(
        _out_body,
        grid=(N // BLK,),
        in_specs=[_rows((BLK, HALF)), _rows((BLK, HALF)), _block((MP, MP)),
                  _block((1, MP)), _block((MP, OUT)), _block((1, OUT))],
        out_specs=_rows((BLK, OUT)),
        out_shape=jax.ShapeDtypeStruct((N, OUT), jnp.float32),
    )(a0, a1, w, b2, wo, bo2)


def kernel(x, edge_index, W_in, b_in, W1, b1, W2, b2, W3, b3, W_out, b_out):
    npad = E_PAD - E
    pad_row = (jnp.arange(npad, dtype=jnp.int32) % NT) * 624
    pad_col = N + (jnp.arange(npad, dtype=jnp.int32) % 8)
    rowp = jnp.concatenate([edge_index[0], pad_row])
    colp = jnp.concatenate([edge_index[1], pad_col])
    idx2 = jnp.stack([rowp.reshape(NCHUNKS, CH), colp.reshape(NCHUNKS, CH)], axis=1)
    col2 = colp.reshape(NCHUNKS, 1, CH)

    d0, d1 = _deg_call(col2)
    deg8 = jnp.broadcast_to((d0 + d1 - 1.0)[:, None], (N, 8))

    g0, g1 = _in_call(x, W_in, b_in.reshape(1, MP), deg8)
    for Wl, bl in ((W1, b1), (W2, b2)):
        a0, a1 = _scatter_call(g0, g1, idx2)
        g0, g1 = _mid_call(a0, a1, Wl, bl.reshape(1, MP), deg8)
    a0, a1 = _scatter_call(g0, g1, idx2)
    return _out_call(a0, a1, W3, b3.reshape(1, MP), W_out, b_out.reshape(1, OUT))


# speedup vs baseline: 15.6661x; 1.0699x over previous
import jax
import jax.numpy as jnp
from jax import lax
from jax.experimental import pallas as pl
from jax.experimental.pallas import tpu as pltpu
from jax.experimental.pallas import tpu_sc as plsc

N = 10000
E = 320000
IN_DIM = 128
MP = 256
HALF = 128
OUT = 128

NT = 16
NC = 2
CH = 128
NCHT = 157
EPT = NCHT * CH
E_PAD = EPT * NT
NCHUNKS = E_PAD // CH
NBUF = 3
NIB = 6
NWB = 79
DNCH = 80
DNCHUNKS = DNCH * NT * NC
DNBUF = 4
DNIB = 8


def _sc_mesh():
    return plsc.VectorSubcoreMesh(core_axis_name="c", subcore_axis_name="s")


def _deg_body(col2_hbm, d0_hbm, d1_hbm, acc_sh, ones_v, idx_v, *sems):
    c = lax.axis_index("c")
    s = lax.axis_index("s")

    def fill_ones(i, _):
        ones_v[pl.ds(i * 16, 16)] = jnp.ones((16,), jnp.float32)
        return 0
    lax.fori_loop(0, 64, fill_ones, 0)

    @pl.when(s < 10)
    def _():
        pltpu.sync_copy(ones_v.at[pl.ds(0, 1000)], acc_sh.at[pl.ds(s * 1000, 1000)])

    @pl.when(s == 10)
    def _():
        pltpu.sync_copy(ones_v.at[pl.ds(0, 8)], acc_sh.at[pl.ds(N, 8)])

    plsc.subcore_barrier()

    sem_i = sems[:DNIB]
    sem_s = sems[DNIB:]
    base = (c * NT + s) * DNCH

    for bi in range(DNIB):
        pltpu.async_copy(col2_hbm.at[base + bi], idx_v.at[bi], sem_i[bi])

    def outer(j, _):
        for bi in range(DNIB):
            b = bi % DNBUF
            i = base + j * DNIB + bi
            pi = (bi - DNBUF) % DNIB

            def drain_and_prefetch(b=b, bi=bi, pi=pi, i=i):
                pltpu.make_async_copy(ones_v.at[pl.ds(0, CH)],
                                      acc_sh.at[idx_v.at[bi, 0]], sem_s[b]).wait()

                @pl.when(i + DNBUF < base + DNCH)
                def _():
                    pltpu.async_copy(col2_hbm.at[i + DNBUF], idx_v.at[pi], sem_i[pi])

            if bi >= DNBUF:
                drain_and_prefetch()
            else:
                @pl.when(j > 0)
                def _(fn=drain_and_prefetch):
                    fn()

            pltpu.make_async_copy(col2_hbm.at[i], idx_v.at[bi], sem_i[bi]).wait()
            pltpu.async_copy(ones_v.at[pl.ds(0, CH)], acc_sh.at[idx_v.at[bi, 0]],
                             sem_s[b], add=True)
        return 0
    lax.fori_loop(0, DNCH // DNIB, outer, 0)

    for b in range(DNBUF):
        bi = DNIB - DNBUF + b
        pltpu.make_async_copy(ones_v.at[pl.ds(0, CH)],
                              acc_sh.at[idx_v.at[bi, 0]], sem_s[b]).wait()

    plsc.subcore_barrier()

    @pl.when(s < 10)
    def _():
        pltpu.sync_copy(acc_sh.at[pl.ds(s * 1000, 1000)], ones_v.at[pl.ds(0, 1000)])

        @pl.when(c == 0)
        def _():
            pltpu.sync_copy(ones_v.at[pl.ds(0, 1000)], d0_hbm.at[pl.ds(s * 1000, 1000)])

        @pl.when(c == 1)
        def _():
            pltpu.sync_copy(ones_v.at[pl.ds(0, 1000)], d1_hbm.at[pl.ds(s * 1000, 1000)])


def _deg_call(col2):
    f = pl.kernel(
        _deg_body,
        out_type=[jax.ShapeDtypeStruct((N,), jnp.float32)] * 2,
        mesh=_sc_mesh(),
        scratch_types=[
            pltpu.VMEM_SHARED((N + 8,), jnp.float32),
            pltpu.VMEM((1024,), jnp.float32),
            pltpu.VMEM((DNIB, 1, CH), jnp.int32),
        ] + [pltpu.SemaphoreType.DMA] * (DNIB + DNBUF),
    )
    return f(col2)


def _scatter_body(g0_hbm, g1_hbm, idx2_hbm, out0_hbm, out1_hbm,
                  acc_sh, idx_v, rows_v, *sems):
    c = lax.axis_index("c")
    s = lax.axis_index("s")

    for k in range(5):
        kk = k * 16 + s
        r0 = kk * CH

        @pl.when(kk < NWB - 1)
        def _(r0=r0):
            @pl.when(c == 0)
            def _():
                pltpu.sync_copy(g0_hbm.at[pl.ds(r0, CH)], acc_sh.at[pl.ds(r0, CH)])

            @pl.when(c == 1)
            def _():
                pltpu.sync_copy(g1_hbm.at[pl.ds(r0, CH)], acc_sh.at[pl.ds(r0, CH)])

        @pl.when(kk == NWB - 1)
        def _():
            @pl.when(c == 0)
            def _():
                pltpu.sync_copy(g0_hbm.at[pl.ds(9984, 16)], acc_sh.at[pl.ds(9984, 16)])

            @pl.when(c == 1)
            def _():
                pltpu.sync_copy(g1_hbm.at[pl.ds(9984, 16)], acc_sh.at[pl.ds(9984, 16)])

    plsc.subcore_barrier()

    sem_i = sems[:NIB]
    sem_g = sems[NIB:NIB + NBUF]
    sem_s = sems[NIB + NBUF:]
    base = s * NCHT

    for bi in range(NIB):
        pltpu.async_copy(idx2_hbm.at[base + bi], idx_v.at[bi], sem_i[bi])

    def _gather(bi, b, i):
        @pl.when(c == 0)
        def _():
            pltpu.async_copy(g0_hbm.at[idx_v.at[bi, 0]], rows_v.at[b], sem_g[b])

        @pl.when(c == 1)
        def _():
            pltpu.async_copy(g1_hbm.at[idx_v.at[bi, 0]], rows_v.at[b], sem_g[b])

        return pltpu.make_async_copy(g0_hbm.at[idx_v.at[bi, 0]], rows_v.at[b],
                                     sem_g[b])

    def outer(j, _):
        for bi in range(NIB):
            b = bi % NBUF
            i = base + j * NIB + bi
            pi = (bi - NBUF) % NIB

            def drain_and_prefetch(b=b, bi=bi, pi=pi, i=i):
                pltpu.make_async_copy(rows_v.at[b], acc_sh.at[idx_v.at[bi, 1]],
                                      sem_s[b]).wait()

                @pl.when(i + NBUF < base + NCHT)
                def _():
                    pltpu.async_copy(idx2_hbm.at[i + NBUF], idx_v.at[pi], sem_i[pi])

            if bi >= NBUF:
                drain_and_prefetch()
            else:
                @pl.when(j > 0)
                def _(fn=drain_and_prefetch):
                    fn()

            pltpu.make_async_copy(idx2_hbm.at[i], idx_v.at[bi], sem_i[bi]).wait()
            _gather(bi, b, i)

            pb, pbuf = (bi - 1) % NIB, (b - 1) % NBUF

            def prev_scatter(pb=pb, pbuf=pbuf):
                pltpu.make_async_copy(g0_hbm.at[idx_v.at[pb, 0]],
                                      rows_v.at[pbuf], sem_g[pbuf]).wait()
                pltpu.async_copy(rows_v.at[pbuf], acc_sh.at[idx_v.at[pb, 1]],
                                 sem_s[pbuf], add=True)

            if bi >= 1:
                prev_scatter()
            else:
                @pl.when(j > 0)
                def _(fn=prev_scatter):
                    fn()

        return 0
    lax.fori_loop(0, NCHT // NIB, outer, 0)

    i_t = base + NCHT - 1
    pltpu.make_async_copy(rows_v.at[0], acc_sh.at[idx_v.at[0, 1]],
                          sem_s[0]).wait()
    pltpu.make_async_copy(idx2_hbm.at[i_t], idx_v.at[0], sem_i[0]).wait()
    _gather(0, 0, i_t)
    pltpu.make_async_copy(g0_hbm.at[idx_v.at[NIB - 1, 0]],
                          rows_v.at[NBUF - 1], sem_g[NBUF - 1]).wait()
    pltpu.async_copy(rows_v.at[NBUF - 1], acc_sh.at[idx_v.at[NIB - 1, 1]],
                     sem_s[NBUF - 1], add=True)
    pltpu.make_async_copy(g0_hbm.at[idx_v.at[0, 0]], rows_v.at[0],
                          sem_g[0]).wait()
    pltpu.async_copy(rows_v.at[0], acc_sh.at[idx_v.at[0, 1]],
                     sem_s[0], add=True)
    for b in range(NBUF):
        pltpu.make_async_copy(rows_v.at[b], acc_sh.at[idx_v.at[b, 1]],
                              sem_s[b]).wait()

    plsc.subcore_barrier()

    for k in range(5):
        kk = k * 16 + s
        r0 = kk * CH

        @pl.when(kk < NWB - 1)
        def _(r0=r0):
            @pl.when(c == 0)
            def _():
                pltpu.sync_copy(acc_sh.at[pl.ds(r0, CH)], out0_hbm.at[pl.ds(r0, CH)])

            @pl.when(c == 1)
            def _():
                pltpu.sync_copy(acc_sh.at[pl.ds(r0, CH)], out1_hbm.at[pl.ds(r0, CH)])

        @pl.when(kk == NWB - 1)
        def _():
            @pl.when(c == 0)
            def _():
                pltpu.sync_copy(acc_sh.at[pl.ds(9984, 16)], out0_hbm.at[pl.ds(9984, 16)])

            @pl.when(c == 1)
            def _():
                pltpu.sync_copy(acc_sh.at[pl.ds(9984, 16)], out1_hbm.at[pl.ds(9984, 16)])


def _scatter_call(g0, g1, idx2):
    f = pl.kernel(
        _scatter_body,
        out_type=[
            jax.ShapeDtypeStruct((N, HALF), jnp.float32),
            jax.ShapeDtypeStruct((N, HALF), jnp.float32),
        ],
        mesh=_sc_mesh(),
        scratch_types=[
            pltpu.VMEM_SHARED((N + 8, HALF), jnp.float32),
            pltpu.VMEM((NIB, 2, CH), jnp.int32),
            pltpu.VMEM((NBUF, CH, HALF), jnp.float32),
        ] + [pltpu.SemaphoreType.DMA] * (NIB + 2 * NBUF),
    )
    return f(g0, g1, idx2)


BLK = 1000


def _gelu(x):
    return x * 0.5 * (1.0 + lax.erf(x * 0.7071067811865476))


def _in_body(x_ref, w_ref, b_ref, deg_ref, g0_ref, g1_ref):
    h = jnp.dot(x_ref[...], w_ref[...], preferred_element_type=jnp.float32)
    h = h + b_ref[...]
    g = _gelu(h) / deg_ref[:, 0:1]
    g0_ref[...] = g[:, :HALF]
    g1_ref[...] = g[:, HALF:]


def _mid_body(a0_ref, a1_ref, w_ref, b_ref, deg_ref, g0_ref, g1_ref):
    h = jnp.dot(a0_ref[...], w_ref[:HALF, :], preferred_element_type=jnp.float32)
    h = h + jnp.dot(a1_ref[...], w_ref[HALF:, :], preferred_element_type=jnp.float32)
    h = h + b_ref[...]
    g = _gelu(h) / deg_ref[:, 0:1]
    g0_ref[...] = g[:, :HALF]
    g1_ref[...] = g[:, HALF:]


def _out_body(a0_ref, a1_ref, w_ref, b_ref, wo_ref, bo_ref, o_ref):
    h = jnp.dot(a0_ref[...], w_ref[:HALF, :], preferred_element_type=jnp.float32)
    h = h + jnp.dot(a1_ref[...], w_ref[HALF:, :], preferred_element_type=jnp.float32)
    h = h + b_ref[...]
    o_ref[...] = jnp.dot(h, wo_ref[...], preferred_element_type=jnp.float32) + bo_ref[...]


def _block(shape):
    return pl.BlockSpec(shape, lambda i: (0,) * len(shape))


def _rows(shape):
    return pl.BlockSpec(shape, lambda i: (i,) + (0,) * (len(shape) - 1))


def _in_call(x, w, b2, deg8):
    return pl.pallas_call(
        _in_body,
        grid=(N // BLK,),
        in_specs=[_rows((BLK, IN_DIM)), _block((IN_DIM, MP)), _block((1, MP)),
                  _rows((BLK, 8))],
        out_specs=[_rows((BLK, HALF)), _rows((BLK, HALF))],
        out_shape=[jax.ShapeDtypeStruct((N, HALF), jnp.float32)] * 2,
    )(x, w, b2, deg8)


def _mid_call(a0, a1, w, b2, deg8):
    return pl.pallas_call(
        _mid_body,
        grid=(N // BLK,),
        in_specs=[_rows((BLK, HALF)), _rows((BLK, HALF)), _block((MP, MP)),
                  _block((1, MP)), _rows((BLK, 8))],
        out_specs=[_rows((BLK, HALF)), _rows((BLK, HALF))],
        out_shape=[jax.ShapeDtypeStruct((N, HALF), jnp.float32)] * 2,
    )(a0, a1, w, b2, deg8)


def _out_call(a0, a1, w, b2, wo, bo2):
    return pl.pallas_call(
        _out_body,
        grid=(N // BLK,),
        in_specs=[_rows((BLK, HALF)), _rows((BLK, HALF)), _block((MP, MP)),
                  _block((1, MP)), _block((MP, OUT)), _block((1, OUT))],
        out_specs=_rows((BLK, OUT)),
        out_shape=jax.ShapeDtypeStruct((N, OUT), jnp.float32),
    )(a0, a1, w, b2, wo, bo2)


def kernel(x, edge_index, W_in, b_in, W1, b1, W2, b2, W3, b3, W_out, b_out):
    npad = E_PAD - E
    pad_row = (jnp.arange(npad, dtype=jnp.int32) % NT) * 624
    pad_col = N + (jnp.arange(npad, dtype=jnp.int32) % 8)
    rowp = jnp.concatenate([edge_index[0], pad_row])
    colp = jnp.concatenate([edge_index[1], pad_col])
    idx2 = jnp.stack([rowp.reshape(NCHUNKS, CH), colp.reshape(NCHUNKS, CH)], axis=1)
    dpad = DNCHUNKS * CH - E
    pad_col_d = N + (jnp.arange(dpad, dtype=jnp.int32) % 8)
    col2 = jnp.concatenate([edge_index[1], pad_col_d]).reshape(DNCHUNKS, 1, CH)

    d0, d1 = _deg_call(col2)
    deg8 = jnp.broadcast_to((d0 + d1 - 1.0)[:, None], (N, 8))

    g0, g1 = _in_call(x, W_in, b_in.reshape(1, MP), deg8)
    for Wl, bl in ((W1, b1), (W2, b2)):
        a0, a1 = _scatter_call(g0, g1, idx2)
        g0, g1 = _mid_call(a0, a1, Wl, bl.reshape(1, MP), deg8)
    a0, a1 = _scatter_call(g0, g1, idx2)
    return _out_call(a0, a1, W3, b3.reshape(1, MP), W_out, b_out.reshape(1, OUT))


# speedup vs baseline: 15.8353x; 1.0108x over previous
import jax
import jax.numpy as jnp
from jax import lax
from jax.experimental import pallas as pl
from jax.experimental.pallas import tpu as pltpu
from jax.experimental.pallas import tpu_sc as plsc

N = 10000
E = 320000
IN_DIM = 128
MP = 256
HALF = 128
OUT = 128

NT = 16
NC = 2
CH = 128
NCHT = 157
EPT = NCHT * CH
E_PAD = EPT * NT
NCHUNKS = E_PAD // CH
NBUF = 3
NIB = 6
NWB = 79
DNCH = 80
DNCHUNKS = DNCH * NT * NC
DNBUF = 4
DNIB = 8


def _sc_mesh():
    return plsc.VectorSubcoreMesh(core_axis_name="c", subcore_axis_name="s")


def _deg_body(col2_hbm, d0_hbm, d1_hbm, acc_sh, ones_v, idx_v, *sems):
    c = lax.axis_index("c")
    s = lax.axis_index("s")

    def fill_ones(i, _):
        ones_v[pl.ds(i * 16, 16)] = jnp.ones((16,), jnp.float32)
        return 0
    lax.fori_loop(0, 64, fill_ones, 0)

    @pl.when(s < 10)
    def _():
        pltpu.sync_copy(ones_v.at[pl.ds(0, 1000)], acc_sh.at[pl.ds(s * 1000, 1000)])

    @pl.when(s == 10)
    def _():
        pltpu.sync_copy(ones_v.at[pl.ds(0, 8)], acc_sh.at[pl.ds(N, 8)])

    plsc.subcore_barrier()

    sem_i = sems[:DNIB]
    sem_s = sems[DNIB:]
    base = (c * NT + s) * DNCH

    for bi in range(DNIB):
        pltpu.async_copy(col2_hbm.at[base + bi], idx_v.at[bi], sem_i[bi])

    def outer(j, _):
        for bi in range(DNIB):
            b = bi % DNBUF
            i = base + j * DNIB + bi
            pi = (bi - DNBUF) % DNIB

            def drain_and_prefetch(b=b, bi=bi, pi=pi, i=i):
                pltpu.make_async_copy(ones_v.at[pl.ds(0, CH)],
                                      acc_sh.at[idx_v.at[bi, 0]], sem_s[b]).wait()

                @pl.when(i + DNBUF < base + DNCH)
                def _():
                    pltpu.async_copy(col2_hbm.at[i + DNBUF], idx_v.at[pi], sem_i[pi])

            if bi >= DNBUF:
                drain_and_prefetch()
            else:
                @pl.when(j > 0)
                def _(fn=drain_and_prefetch):
                    fn()

            pltpu.make_async_copy(col2_hbm.at[i], idx_v.at[bi], sem_i[bi]).wait()
            pltpu.async_copy(ones_v.at[pl.ds(0, CH)], acc_sh.at[idx_v.at[bi, 0]],
                             sem_s[b], add=True)
        return 0
    lax.fori_loop(0, DNCH // DNIB, outer, 0)

    for b in range(DNBUF):
        bi = DNIB - DNBUF + b
        pltpu.make_async_copy(ones_v.at[pl.ds(0, CH)],
                              acc_sh.at[idx_v.at[bi, 0]], sem_s[b]).wait()

    plsc.subcore_barrier()

    @pl.when(s < 10)
    def _():
        pltpu.sync_copy(acc_sh.at[pl.ds(s * 1000, 1000)], ones_v.at[pl.ds(0, 1000)])

        @pl.when(c == 0)
        def _():
            pltpu.sync_copy(ones_v.at[pl.ds(0, 1000)], d0_hbm.at[pl.ds(s * 1000, 1000)])

        @pl.when(c == 1)
        def _():
            pltpu.sync_copy(ones_v.at[pl.ds(0, 1000)], d1_hbm.at[pl.ds(s * 1000, 1000)])


def _deg_call(col2):
    f = pl.kernel(
        _deg_body,
        out_type=[jax.ShapeDtypeStruct((N,), jnp.float32)] * 2,
        mesh=_sc_mesh(),
        scratch_types=[
            pltpu.VMEM_SHARED((N + 8,), jnp.float32),
            pltpu.VMEM((1024,), jnp.float32),
            pltpu.VMEM((DNIB, 1, CH), jnp.int32),
        ] + [pltpu.SemaphoreType.DMA] * (DNIB + DNBUF),
    )
    return f(col2)


def _scatter_body(g0_hbm, g1_hbm, idx2_hbm, out0_hbm, out1_hbm,
                  acc_sh, idx_v, rows_v, *sems):
    c = lax.axis_index("c")
    s = lax.axis_index("s")

    for k in range(5):
        kk = k * 16 + s
        r0 = kk * CH

        @pl.when(kk < NWB - 1)
        def _(r0=r0):
            @pl.when(c == 0)
            def _():
                pltpu.sync_copy(g0_hbm.at[pl.ds(r0, CH)], acc_sh.at[pl.ds(r0, CH)])

            @pl.when(c == 1)
            def _():
                pltpu.sync_copy(g1_hbm.at[pl.ds(r0, CH)], acc_sh.at[pl.ds(r0, CH)])

        @pl.when(kk == NWB - 1)
        def _():
            @pl.when(c == 0)
            def _():
                pltpu.sync_copy(g0_hbm.at[pl.ds(9984, 16)], acc_sh.at[pl.ds(9984, 16)])

            @pl.when(c == 1)
            def _():
                pltpu.sync_copy(g1_hbm.at[pl.ds(9984, 16)], acc_sh.at[pl.ds(9984, 16)])

    plsc.subcore_barrier()

    sem_i = sems[:NIB]
    sem_g = sems[NIB:NIB + NBUF]
    sem_s = sems[NIB + NBUF:]
    base = s * NCHT

    for bi in range(NIB):
        pltpu.async_copy(idx2_hbm.at[base + bi], idx_v.at[bi], sem_i[bi])

    def _gather(bi, b, i):
        @pl.when(c == 0)
        def _():
            pltpu.async_copy(g0_hbm.at[idx_v.at[bi, 0]], rows_v.at[b], sem_g[b])

        @pl.when(c == 1)
        def _():
            pltpu.async_copy(g1_hbm.at[idx_v.at[bi, 0]], rows_v.at[b], sem_g[b])

        return pltpu.make_async_copy(g0_hbm.at[idx_v.at[bi, 0]], rows_v.at[b],
                                     sem_g[b])

    def outer(j, _):
        for bi in range(NIB):
            b = bi % NBUF
            i = base + j * NIB + bi
            pi = (bi - NBUF) % NIB

            def drain_and_prefetch(b=b, bi=bi, pi=pi, i=i):
                pltpu.make_async_copy(rows_v.at[b], acc_sh.at[idx_v.at[bi, 1]],
                                      sem_s[b]).wait()

                @pl.when(i + NBUF < base + NCHT)
                def _():
                    pltpu.async_copy(idx2_hbm.at[i + NBUF], idx_v.at[pi], sem_i[pi])

            if bi >= NBUF:
                drain_and_prefetch()
            else:
                @pl.when(j > 0)
                def _(fn=drain_and_prefetch):
                    fn()

            pltpu.make_async_copy(idx2_hbm.at[i], idx_v.at[bi], sem_i[bi]).wait()
            _gather(bi, b, i)

            pb, pbuf = (bi - 1) % NIB, (b - 1) % NBUF

            def prev_scatter(pb=pb, pbuf=pbuf):
                pltpu.make_async_copy(g0_hbm.at[idx_v.at[pb, 0]],
                                      rows_v.at[pbuf], sem_g[pbuf]).wait()
                pltpu.async_copy(rows_v.at[pbuf], acc_sh.at[idx_v.at[pb, 1]],
                                 sem_s[pbuf], add=True)

            if bi >= 1:
                prev_scatter()
            else:
                @pl.when(j > 0)
                def _(fn=prev_scatter):
                    fn()

        return 0
    lax.fori_loop(0, NCHT // NIB, outer, 0)

    i_t = base + NCHT - 1
    pltpu.make_async_copy(rows_v.at[0], acc_sh.at[idx_v.at[0, 1]],
                          sem_s[0]).wait()
    pltpu.make_async_copy(idx2_hbm.at[i_t], idx_v.at[0], sem_i[0]).wait()
    _gather(0, 0, i_t)
    pltpu.make_async_copy(g0_hbm.at[idx_v.at[NIB - 1, 0]],
                          rows_v.at[NBUF - 1], sem_g[NBUF - 1]).wait()
    pltpu.async_copy(rows_v.at[NBUF - 1], acc_sh.at[idx_v.at[NIB - 1, 1]],
                     sem_s[NBUF - 1], add=True)
    pltpu.make_async_copy(g0_hbm.at[idx_v.at[0, 0]], rows_v.at[0],
                          sem_g[0]).wait()
    pltpu.async_copy(rows_v.at[0], acc_sh.at[idx_v.at[0, 1]],
                     sem_s[0], add=True)
    for b in range(NBUF):
        pltpu.make_async_copy(rows_v.at[b], acc_sh.at[idx_v.at[b, 1]],
                              sem_s[b]).wait()

    plsc.subcore_barrier()

    for k in range(5):
        kk = k * 16 + s
        r0 = kk * CH

        @pl.when(kk < NWB - 1)
        def _(r0=r0):
            @pl.when(c == 0)
            def _():
                pltpu.sync_copy(acc_sh.at[pl.ds(r0, CH)], out0_hbm.at[pl.ds(r0, CH)])

            @pl.when(c == 1)
            def _():
                pltpu.sync_copy(acc_sh.at[pl.ds(r0, CH)], out1_hbm.at[pl.ds(r0, CH)])

        @pl.when(kk == NWB - 1)
        def _():
            @pl.when(c == 0)
            def _():
                pltpu.sync_copy(acc_sh.at[pl.ds(9984, 16)], out0_hbm.at[pl.ds(9984, 16)])

            @pl.when(c == 1)
            def _():
                pltpu.sync_copy(acc_sh.at[pl.ds(9984, 16)], out1_hbm.at[pl.ds(9984, 16)])


def _scatter_call(g0, g1, idx2):
    f = pl.kernel(
        _scatter_body,
        out_type=[
            jax.ShapeDtypeStruct((N, HALF), jnp.float32),
            jax.ShapeDtypeStruct((N, HALF), jnp.float32),
        ],
        mesh=_sc_mesh(),
        scratch_types=[
            pltpu.VMEM_SHARED((N + 8, HALF), jnp.float32),
            pltpu.VMEM((NIB, 2, CH), jnp.int32),
            pltpu.VMEM((NBUF, CH, HALF), jnp.float32),
        ] + [pltpu.SemaphoreType.DMA] * (NIB + 2 * NBUF),
    )
    return f(g0, g1, idx2)


BLK = 2000


def _gelu(x):
    return x * 0.5 * (1.0 + lax.erf(x * 0.7071067811865476))


def _in_body(x_ref, w_ref, b_ref, deg_ref, g0_ref, g1_ref):
    h = jnp.dot(x_ref[...], w_ref[...], preferred_element_type=jnp.float32)
    h = h + b_ref[...]
    g = _gelu(h) / deg_ref[:, 0:1]
    g0_ref[...] = g[:, :HALF]
    g1_ref[...] = g[:, HALF:]


def _mid_body(a0_ref, a1_ref, w_ref, b_ref, deg_ref, g0_ref, g1_ref):
    h = jnp.dot(a0_ref[...], w_ref[:HALF, :], preferred_element_type=jnp.float32)
    h = h + jnp.dot(a1_ref[...], w_ref[HALF:, :], preferred_element_type=jnp.float32)
    h = h + b_ref[...]
    g = _gelu(h) / deg_ref[:, 0:1]
    g0_ref[...] = g[:, :HALF]
    g1_ref[...] = g[:, HALF:]


def _out_body(a0_ref, a1_ref, w_ref, b_ref, wo_ref, bo_ref, o_ref):
    h = jnp.dot(a0_ref[...], w_ref[:HALF, :], preferred_element_type=jnp.float32)
    h = h + jnp.dot(a1_ref[...], w_ref[HALF:, :], preferred_element_type=jnp.float32)
    h = h + b_ref[...]
    o_ref[...] = jnp.dot(h, wo_ref[...], preferred_element_type=jnp.float32) + bo_ref[...]


def _block(shape):
    return pl.BlockSpec(shape, lambda i: (0,) * len(shape))


def _rows(shape):
    return pl.BlockSpec(shape, lambda i: (i,) + (0,) * (len(shape) - 1))


def _in_call(x, w, b2, deg8):
    return pl.pallas_call(
        _in_body,
        grid=(N // BLK,),
        in_specs=[_rows((BLK, IN_DIM)), _block((IN_DIM, MP)), _block((1, MP)),
                  _rows((BLK, 8))],
        out_specs=[_rows((BLK, HALF)), _rows((BLK, HALF))],
        out_shape=[jax.ShapeDtypeStruct((N, HALF), jnp.float32)] * 2,
    )(x, w, b2, deg8)


def _mid_call(a0, a1, w, b2, deg8):
    return pl.pallas_call(
        _mid_body,
        grid=(N // BLK,),
        in_specs=[_rows((BLK, HALF)), _rows((BLK, HALF)), _block((MP, MP)),
                  _block((1, MP)), _rows((BLK, 8))],
        out_specs=[_rows((BLK, HALF)), _rows((BLK, HALF))],
        out_shape=[jax.ShapeDtypeStruct((N, HALF), jnp.float32)] * 2,
    )(a0, a1, w, b2, deg8)


def _out_call(a0, a1, w, b2, wo, bo2):
    return pl.pallas_call(
        _out_body,
        grid=(N // BLK,),
        in_specs=[_rows((BLK, HALF)), _rows((BLK, HALF)), _block((MP, MP)),
                  _block((1, MP)), _block((MP, OUT)), _block((1, OUT))],
        out_specs=_rows((BLK, OUT)),
        out_shape=jax.ShapeDtypeStruct((N, OUT), jnp.float32),
    )(a0, a1, w, b2, wo, bo2)


def kernel(x, edge_index, W_in, b_in, W1, b1, W2, b2, W3, b3, W_out, b_out):
    npad = E_PAD - E
    pad_row = (jnp.arange(npad, dtype=jnp.int32) % NT) * 624
    pad_col = N + (jnp.arange(npad, dtype=jnp.int32) % 8)
    rowp = jnp.concatenate([edge_index[0], pad_row])
    colp = jnp.concatenate([edge_index[1], pad_col])
    idx2 = jnp.stack([rowp.reshape(NCHUNKS, CH), colp.reshape(NCHUNKS, CH)], axis=1)
    dpad = DNCHUNKS * CH - E
    pad_col_d = N + (jnp.arange(dpad, dtype=jnp.int32) % 8)
    col2 = jnp.concatenate([edge_index[1], pad_col_d]).reshape(DNCHUNKS, 1, CH)

    d0, d1 = _deg_call(col2)
    deg8 = jnp.broadcast_to((d0 + d1 - 1.0)[:, None], (N, 8))

    g0, g1 = _in_call(x, W_in, b_in.reshape(1, MP), deg8)
    for Wl, bl in ((W1, b1), (W2, b2)):
        a0, a1 = _scatter_call(g0, g1, idx2)
        g0, g1 = _mid_call(a0, a1, Wl, bl.reshape(1, MP), deg8)
    a0, a1 = _scatter_call(g0, g1, idx2)
    return _out_call(a0, a1, W3, b3.reshape(1, MP), W_out, b_out.reshape(1, OUT))
